# baseline (speedup 1.0000x reference)
import functools

import jax
import jax.numpy as jnp
from jax import lax
from jax.experimental import pallas as pl
from jax.experimental.pallas import tpu as pltpu
from jax.experimental.pallas import tpu_sc as plsc

B = 64
N = 1024
K = 512
D = 96
N1 = N + 1
OUT_ROWS = K + 1

_R = 8
_JC = 128

_NC = 2
_NS = 16
_NW = _NC * _NS
_ROWS_PER_W = B // _NW

_PAD_SLOTS = 528
_CHUNK = 88
_NCHUNK = _PAD_SLOTS // _CHUNK


def _rank_body(rand_ref, rank_ref):
    v = rand_ref[...]
    a = v[:, :, None]
    ii = lax.broadcasted_iota(jnp.int32, (N, _JC), 0)
    jj0 = lax.broadcasted_iota(jnp.int32, (N, _JC), 1)

    acc = jnp.zeros((_R, N), jnp.int32)
    for jc in range(N // _JC):
        vb = v[:, jc * _JC:(jc + 1) * _JC]
        jlt = (jc * _JC + jj0) < ii
        b = vb[:, None, :]
        cnt = (b > a) | ((b == a) & jlt[None])
        acc = acc + jnp.sum(cnt.astype(jnp.int32), axis=2)
    rank_ref[...] = acc


@jax.jit
def _ranks_tc(rand):
    return pl.pallas_call(
        _rank_body,
        grid=(B // _R,),
        in_specs=[pl.BlockSpec((_R, N), lambda g: (g, 0))],
        out_specs=pl.BlockSpec((_R, N), lambda g: (g, 0)),
        out_shape=jax.ShapeDtypeStruct((B, N), jnp.int32),
    )(rand)


def _sc_body(x_hbm, ranks_hbm, out_hbm, rank_v, keep_v, rows_v, sem):
    wid = lax.axis_index("s") * _NC + lax.axis_index("c")
    iota = lax.iota(jnp.int32, 16)

    def row_body(t, carry):
        b = wid * _ROWS_PER_W + t
        pltpu.sync_copy(ranks_hbm.at[pl.ds(b * N, N)], rank_v)

        def zinit(g, c):
            keep_v[pl.ds(g * 16, 16)] = jnp.zeros((16,), jnp.int32)
            return c
        lax.fori_loop(0, _PAD_SLOTS // 16, zinit, 0)
        keep_v[pl.ds(0, 16)] = jnp.where(iota == 0, b * N1, 0)

        base_val = b * N1 + 1

        def scat(g, c):
            r = rank_v[pl.ds(g * 16, 16)]
            pos = jnp.minimum(r + 1, _PAD_SLOTS - 1)
            val = base_val + g * 16 + iota
            plsc.store_scatter(keep_v, [pos], val, mask=r < K)
            return c
        lax.fori_loop(0, N // 16, scat, 0)

        copies = [
            pltpu.async_copy(
                x_hbm.at[keep_v.at[pl.ds(c * _CHUNK, _CHUNK)]],
                rows_v.at[pl.ds(c * _CHUNK, _CHUNK)],
                sem,
            )
            for c in range(_NCHUNK)
        ]
        for cp in copies:
            cp.wait()
        pltpu.sync_copy(rows_v.at[pl.ds(0, OUT_ROWS)], out_hbm.at[b])
        return carry

    lax.fori_loop(0, _ROWS_PER_W, row_body, 0)


@jax.jit
def _gather_sc(x2d, ranks_flat):
    mesh = plsc.VectorSubcoreMesh(core_axis_name="c", subcore_axis_name="s")
    run = functools.partial(
        pl.kernel,
        mesh=mesh,
        out_type=jax.ShapeDtypeStruct((B, OUT_ROWS, D), jnp.float32),
        scratch_types=[
            pltpu.VMEM((N,), jnp.int32),
            pltpu.VMEM((_PAD_SLOTS,), jnp.int32),
            pltpu.VMEM((_PAD_SLOTS, D), jnp.float32),
            pltpu.SemaphoreType.DMA,
        ],
        compiler_params=pltpu.CompilerParams(
            needs_layout_passes=False, use_tc_tiling_on_sc=False),
    )(_sc_body)
    return run(x2d, ranks_flat)


def kernel(x, rand):
    ranks = _ranks_tc(rand)
    x2d = x.reshape(B * N1, D)
    return _gather_sc(x2d, ranks.reshape(-1))
